# baseline (speedup 1.0000x reference)
import functools

import numpy as np
import jax
import jax.numpy as jnp
from jax import lax
from jax.experimental import pallas as pl
from jax.experimental.pallas import tpu as pltpu
from jax.experimental.pallas import tpu_sc as plsc

N = 10000
E = 320000
EMB = 128
HID = 128
HEADS = 8
D = HEADS * HID

NPAD = 10240
BM = 1024

NC = 2
NS = 16
NW = NC * NS
EC = E // NW
BLK = 80
NBLK = EC // BLK

NRANGE = 14
RNG = 768
RPT = RNG // NS
CAP = 1088
G = 16
NSLOT = 4

_i32 = jnp.int32
_f32 = jnp.float32

_SEL = np.zeros((D, 16), dtype=np.float32)
for _h in range(HEADS):
    _SEL[_h * HID:(_h + 1) * HID, _h] = 1.0

@functools.cache
def _mesh():
    return plsc.VectorSubcoreMesh(core_axis_name="c", subcore_axis_name="s",
                                  num_cores=NC, num_subcores=NS)


def _linear_attn_body(act, x_ref, b_ref, w_ref, af_src_ref, af_dst_ref,
                      sel_ref, h_ref, s_ref, d_ref):
    x = x_ref[...]
    if act:
        x = jnp.tanh(x + b_ref[...])
    h = jnp.dot(x, w_ref[...], preferred_element_type=jnp.float32)
    h_ref[...] = h
    sel = sel_ref[...]
    s_ref[...] = jnp.dot(h * af_src_ref[...], sel,
                         preferred_element_type=jnp.float32)
    d_ref[...] = jnp.dot(h * af_dst_ref[...], sel,
                         preferred_element_type=jnp.float32)


def _tc_linear_attn(x_pad, bias, W, a_src, a_dst, act):
    k = x_pad.shape[1]
    af_src = a_src.reshape(1, D)
    af_dst = a_dst.reshape(1, D)
    sel = jnp.asarray(_SEL)
    b2d = bias.reshape(1, k) if act else jnp.zeros((1, k), _f32)
    grid = NPAD // BM
    h, s, d = pl.pallas_call(
        functools.partial(_linear_attn_body, act),
        grid=(grid,),
        in_specs=[
            pl.BlockSpec((BM, k), lambda i: (i, 0)),
            pl.BlockSpec((1, k), lambda i: (0, 0)),
            pl.BlockSpec((k, D), lambda i: (0, 0)),
            pl.BlockSpec((1, D), lambda i: (0, 0)),
            pl.BlockSpec((1, D), lambda i: (0, 0)),
            pl.BlockSpec((D, 16), lambda i: (0, 0)),
        ],
        out_specs=[
            pl.BlockSpec((BM, D), lambda i: (i, 0)),
            pl.BlockSpec((BM, 16), lambda i: (i, 0)),
            pl.BlockSpec((BM, 16), lambda i: (i, 0)),
        ],
        out_shape=[
            jax.ShapeDtypeStruct((NPAD, D), _f32),
            jax.ShapeDtypeStruct((NPAD, 16), _f32),
            jax.ShapeDtypeStruct((NPAD, 16), _f32),
        ],
    )(x_pad, b2d, W, af_src, af_dst, sel)
    return h, s, d


def _recip_body(a_ref, b_ref, o_ref):
    o_ref[...] = 1.0 / (a_ref[...] + b_ref[...] + 1e-16)


def _tc_recip(denp):
    a = denp[:NPAD].reshape(1280, 128)
    b = denp[NPAD:].reshape(1280, 128)
    out = pl.pallas_call(
        _recip_body,
        out_shape=jax.ShapeDtypeStruct((1280, 128), _f32),
    )(a, b)
    return out.reshape(NPAD, 16)


def _final_body(x_ref, b_ref, wl_ref, o_ref):
    x = jnp.tanh(x_ref[...] + b_ref[...])
    o_ref[...] = jnp.dot(x, wl_ref[...], preferred_element_type=jnp.float32)


def _tc_final(pre, bias, W_lin):
    wl = jnp.zeros((D, 128), _f32).at[:, :1].set(W_lin)
    b2d = bias.reshape(1, D)
    out = pl.pallas_call(
        _final_body,
        grid=(NPAD // BM,),
        in_specs=[
            pl.BlockSpec((BM, D), lambda i: (i, 0)),
            pl.BlockSpec((1, D), lambda i: (0, 0)),
            pl.BlockSpec((D, 128), lambda i: (0, 0)),
        ],
        out_specs=pl.BlockSpec((BM, 128), lambda i: (i, 0)),
        out_shape=jax.ShapeDtypeStruct((NPAD, 128), _f32),
    )(pre, b2d, wl)
    return out


def _edge_a_body(s_tab, d_tab, src_hbm, dst_hbm,
                 ex_hbm, denp_hbm, eid_hbm, srcb_hbm, dstgb_hbm, cnt_hbm,
                 src_v, dst_v, s_rows, d_rows, bk_eid, bk_src, bk_dstg,
                 zeros_v, idx_scr, dsti, den_sh, sem_g1, sem_g2, sem_f,
                 sem_e):
    c = lax.axis_index("c")
    s = lax.axis_index("s")
    wid = s * NC + c
    ebase = wid * EC

    pltpu.sync_copy(src_hbm.at[pl.ds(ebase, EC)], src_v)
    pltpu.sync_copy(dst_hbm.at[pl.ds(ebase, EC)], dst_v)

    zvec = jnp.zeros((16,), _f32)
    for i in range(128):
        zeros_v[i, :] = zvec
    for r in range(5):
        pltpu.sync_copy(zeros_v, den_sh.at[pl.ds(s * 640 + r * 128, 128)])
    plsc.subcore_barrier()

    lane = lax.iota(_i32, 16)

    def do_block(i, sl, offs):
        eb = i * BLK
        pltpu.make_async_copy(s_tab.at[src_v.at[pl.ds(eb, BLK)]],
                              s_rows.at[sl], sem_g1.at[sl]).wait()
        pltpu.make_async_copy(d_tab.at[dst_v.at[pl.ds(eb, BLK)]],
                              d_rows.at[sl], sem_g2.at[sl]).wait()

        def sub(st, offs):
            sb = st * 16
            for r in range(16):
                idx = sb + r
                ev = s_rows[sl, idx, :] + d_rows[sl, idx, :]
                ev = jnp.where(ev >= 0.0, ev, 0.2 * ev)
                s_rows[sl, idx, :] = jnp.exp(ev)
            dstv = dst_v[pl.ds(eb + sb, 16)]
            dsti[sl, pl.ds(sb, 16)] = dstv
            srcv = src_v[pl.ds(eb + sb, 16)]
            eidv = jnp.full((16,), ebase + eb + sb, _i32) + lane
            new_offs = []
            for b in range(NRANGE):
                lo = b * RNG
                m = (dstv >= lo) & (dstv < lo + RNG)
                cnt = jnp.max(plsc.all_reduce_population_count(m))
                rel = offs[b]
                addr = b * CAP + rel
                plsc.store_compressed(bk_eid.at[pl.ds(addr, 16)], eidv,
                                      mask=m)
                plsc.store_compressed(bk_src.at[pl.ds(addr, 16)], srcv,
                                      mask=m)
                plsc.store_compressed(bk_dstg.at[pl.ds(addr, 16)], dstv,
                                      mask=m)
                new_offs.append(jnp.minimum(rel + cnt, CAP - 64))
            return tuple(new_offs)

        offs = lax.fori_loop(0, BLK // 16, sub, offs)
        pltpu.async_copy(s_rows.at[sl], den_sh.at[dsti.at[sl]],
                         sem_f.at[sl], add=True)
        pltpu.async_copy(s_rows.at[sl], ex_hbm.at[pl.ds(ebase + eb, BLK)],
                         sem_e.at[sl])
        return offs

    def issue_blk(j, sl):
        eb = j * BLK
        pltpu.async_copy(s_tab.at[src_v.at[pl.ds(eb, BLK)]],
                         s_rows.at[sl], sem_g1.at[sl])
        pltpu.async_copy(d_tab.at[dst_v.at[pl.ds(eb, BLK)]],
                         d_rows.at[sl], sem_g2.at[sl])

    def wait_flush(sl):
        pltpu.make_async_copy(s_rows.at[sl], den_sh.at[dsti.at[sl]],
                              sem_f.at[sl]).wait()
        pltpu.make_async_copy(s_rows.at[sl], ex_hbm.at[pl.ds(0, BLK)],
                              sem_e.at[sl]).wait()

    issue_blk(0, 0)
    issue_blk(1, 1)

    def group(i3, offs):
        for sl in range(3):
            i = i3 * 3 + sl
            offs = do_block(i, sl, offs)
            so2 = (sl + 2) % 3

            @pl.when(i >= 1)
            def _wf(so2=so2):
                wait_flush(so2)

            issue_blk(i + 2, so2)
        return offs

    offs = lax.fori_loop(0, (NBLK - 2) // 3, group,
                         tuple(jnp.zeros((), _i32) for _ in range(NRANGE)))
    offs = do_block(jnp.int32(NBLK - 2), 0, offs)
    offs = do_block(jnp.int32(NBLK - 1), 1, offs)
    for sl in (2, 0, 1):
        wait_flush(sl)

    zi = jnp.zeros((16,), _i32)
    ze = jnp.full((16,), E, _i32)
    for b in range(NRANGE):
        padd = jnp.full((16,), b * RNG, _i32)
        for t in range(3):
            o = b * CAP + offs[b] + t * 16
            bk_eid[pl.ds(o, 16)] = ze
            bk_src[pl.ds(o, 16)] = zi
            bk_dstg[pl.ds(o, 16)] = padd
    cv = jnp.zeros((16,), _i32)
    for b in range(NRANGE):
        cv = jnp.where(lane == b, jnp.full((16,), 1, _i32) * offs[b], cv)
    idx_scr[...] = cv
    pltpu.sync_copy(idx_scr, cnt_hbm.at[pl.ds(wid * 16, 16)])
    for b in range(NRANGE):
        seg = (wid * NRANGE + b) * CAP
        pltpu.sync_copy(bk_eid.at[pl.ds(b * CAP, CAP)],
                        eid_hbm.at[pl.ds(seg, CAP)])
        pltpu.sync_copy(bk_src.at[pl.ds(b * CAP, CAP)],
                        srcb_hbm.at[pl.ds(seg, CAP)])
        pltpu.sync_copy(bk_dstg.at[pl.ds(b * CAP, CAP)],
                        dstgb_hbm.at[pl.ds(seg, CAP)])

    plsc.subcore_barrier()
    pltpu.sync_copy(den_sh.at[pl.ds(s * 640, 640)],
                    denp_hbm.at[pl.ds(c * NPAD + s * 640, 640)])


@functools.cache
def _edge_a():
    return pl.kernel(
        _edge_a_body,
        out_type=[
            jax.ShapeDtypeStruct((E, 16), _f32),
            jax.ShapeDtypeStruct((NC * NPAD, 16), _f32),
            jax.ShapeDtypeStruct((NW * NRANGE * CAP,), _i32),
            jax.ShapeDtypeStruct((NW * NRANGE * CAP,), _i32),
            jax.ShapeDtypeStruct((NW * NRANGE * CAP,), _i32),
            jax.ShapeDtypeStruct((NW * 16,), _i32),
        ],
        mesh=_mesh(),
        compiler_params=pltpu.CompilerParams(
            needs_layout_passes=False, use_tc_tiling_on_sc=False),
        scratch_types=[
            pltpu.VMEM((EC,), _i32),
            pltpu.VMEM((EC,), _i32),
            pltpu.VMEM((3, BLK, 16), _f32),
            pltpu.VMEM((3, BLK, 16), _f32),
            pltpu.VMEM((NRANGE * CAP,), _i32),
            pltpu.VMEM((NRANGE * CAP,), _i32),
            pltpu.VMEM((NRANGE * CAP,), _i32),
            pltpu.VMEM((128, 16), _f32),
            pltpu.VMEM((16,), _i32),
            pltpu.VMEM((3, BLK), _i32),
            pltpu.VMEM_SHARED((NPAD, 16), _f32),
            pltpu.SemaphoreType.DMA((3,)),
            pltpu.SemaphoreType.DMA((3,)),
            pltpu.SemaphoreType.DMA((3,)),
            pltpu.SemaphoreType.DMA((3,)),
        ],
    )


def _wmul_body(ex_hbm, denp_hbm, dst_hbm, w_hbm, exb, dnb, dn2, dst_v,
               dst2_v, sem1, sem2, sem3):
    c = lax.axis_index("c")
    s = lax.axis_index("s")
    wid = s * NC + c
    ebase = wid * EC
    pltpu.sync_copy(dst_hbm.at[pl.ds(ebase, EC)], dst_v)
    npadv = jnp.full((16,), NPAD, _i32)

    def shift(i, _):
        dst2_v[pl.ds(i * 16, 16)] = dst_v[pl.ds(i * 16, 16)] + npadv
        return 0

    lax.fori_loop(0, EC // 16, shift, 0)

    def blk_body(blk, _):
        eb = blk * BLK
        cp1 = pltpu.async_copy(ex_hbm.at[pl.ds(ebase + eb, BLK)], exb, sem1)
        cp2 = pltpu.async_copy(denp_hbm.at[dst_v.at[pl.ds(eb, BLK)]],
                               dnb, sem2)
        cp3 = pltpu.async_copy(denp_hbm.at[dst2_v.at[pl.ds(eb, BLK)]],
                               dn2, sem3)
        cp1.wait()
        cp2.wait()
        cp3.wait()
        for r in range(BLK):
            exb[r, :] = exb[r, :] / (dnb[r, :] + dn2[r, :] + 1e-16)
        pltpu.sync_copy(exb, w_hbm.at[pl.ds(ebase + eb, BLK)])
        return 0

    lax.fori_loop(0, NBLK, blk_body, 0)

    @pl.when(wid == 0)
    def _zero_tail():
        for r in range(16):
            dnb[r, :] = jnp.zeros((16,), _f32)
        pltpu.sync_copy(dnb.at[pl.ds(0, 16)], w_hbm.at[pl.ds(E, 16)])


@functools.cache
def _wmul():
    return pl.kernel(
        _wmul_body,
        out_type=jax.ShapeDtypeStruct((E + 16, 16), _f32),
        mesh=_mesh(),
        compiler_params=pltpu.CompilerParams(
            needs_layout_passes=False, use_tc_tiling_on_sc=False),
        scratch_types=[
            pltpu.VMEM((BLK, 16), _f32),
            pltpu.VMEM((BLK, 16), _f32),
            pltpu.VMEM((BLK, 16), _f32),
            pltpu.VMEM((EC,), _i32),
            pltpu.VMEM((EC,), _i32),
            pltpu.SemaphoreType.DMA,
            pltpu.SemaphoreType.DMA,
            pltpu.SemaphoreType.DMA,
        ],
    )


def _agg_body(h_hbm, w_hbm, eidb_hbm, srcb_hbm, dstgb_hbm,
              cnt_hbm, agg_hbm,
              eid_v, src_v, dstg_v, cnt_v, h_buf, exw, w_scr, dli,
              zer, acc_sh, sem_gh, sem_ge, sem_s):
    c = lax.axis_index("c")
    s = lax.axis_index("s")
    lane = lax.iota(_i32, 16)

    zvec = jnp.zeros((16,), _f32)
    for i in range(2):
        for j in range(64):
            zer[i, pl.ds(j * 16, 16)] = zvec

    def pass_body(p, _):
        b = p * NC + c
        rowbase = b * RNG

        def zero_body(r, _):
            pltpu.sync_copy(zer, acc_sh.at[pl.ds(s * RPT + r * 2, 2)])
            return 0

        lax.fori_loop(0, RPT // 2, zero_body, 0)
        plsc.subcore_barrier()

        def chunk_body(slot, _):
            chunk = s * 2 + slot
            seg = (chunk * NRANGE + b) * CAP
            pltpu.sync_copy(eidb_hbm.at[pl.ds(seg, CAP)], eid_v)
            pltpu.sync_copy(srcb_hbm.at[pl.ds(seg, CAP)], src_v)
            pltpu.sync_copy(dstgb_hbm.at[pl.ds(seg, CAP)], dstg_v)
            pltpu.sync_copy(cnt_hbm.at[pl.ds(chunk * 16, 16)], cnt_v)
            bspl = jnp.full((16,), 1, _i32) * b
            count = jnp.max(jnp.where(lane == bspl, cnt_v[...], 0))
            nb = (count + (G - 1)) >> 4
            rb = jnp.full((16,), 1, _i32) * rowbase

            def issue(j, sl):
                base = j * G
                pltpu.async_copy(h_hbm.at[src_v.at[pl.ds(base, G)]],
                                 h_buf.at[sl], sem_gh.at[sl])
                pltpu.async_copy(w_hbm.at[eid_v.at[pl.ds(base, G)]],
                                 exw.at[sl], sem_ge.at[sl])

            @pl.when(nb > 0)
            def _prime0():
                issue(0, 0)

            @pl.when(nb > 1)
            def _prime1():
                issue(1, 1)

            @pl.when(nb > 2)
            def _prime2():
                issue(2, 2)

            def group(i3, _):
                for sl in range(NSLOT):
                    i = i3 * NSLOT + sl

                    @pl.when(i < nb)
                    def _do(i=i, sl=sl):
                        base = i * G
                        pltpu.make_async_copy(
                            h_hbm.at[src_v.at[pl.ds(base, G)]],
                            h_buf.at[sl], sem_gh.at[sl]).wait()
                        pltpu.make_async_copy(
                            w_hbm.at[eid_v.at[pl.ds(base, G)]],
                            exw.at[sl], sem_ge.at[sl]).wait()
                        dli[sl, pl.ds(0, 16)] = (
                            dstg_v[pl.ds(base, 16)] - rb)

                        def edge(g2, _):
                            for e in range(2):
                                g = g2 * 2 + e
                                wv = exw[sl, g, :]
                                w_scr[pl.ds(e * 32, 16)] = wv
                                w_scr[pl.ds(e * 32 + 16, 16)] = wv
                            for e in range(2):
                                g = g2 * 2 + e
                                for hh in range(HEADS):
                                    spl = plsc.load_gather(
                                        w_scr,
                                        [jnp.full((16,),
                                                  e * 32 + 16 + hh, _i32)])
                                    for jj in range(8):
                                        off = hh * HID + jj * 16
                                        h_buf[sl, g, pl.ds(off, 16)] = (
                                            h_buf[sl, g, pl.ds(off, 16)]
                                            * spl)
                            return 0

                        lax.fori_loop(0, G // 2, edge, 0)
                        pltpu.async_copy(h_buf.at[sl],
                                         acc_sh.at[dli.at[sl]],
                                         sem_s.at[sl], add=True)
                        jn = i + 3
                        so2 = (sl + 3) % NSLOT

                        @pl.when(jn < nb)
                        def _prep():
                            @pl.when(i >= 1)
                            def _wait_prev():
                                pltpu.make_async_copy(
                                    h_buf.at[so2],
                                    acc_sh.at[dli.at[so2]],
                                    sem_s.at[so2]).wait()
                            issue(jn, so2)
                return 0

            nb3 = (nb + (NSLOT - 1)) // NSLOT
            lax.fori_loop(0, nb3, group, 0)
            for sl in range(NSLOT):
                @pl.when(sl < nb)
                def _drain(sl=sl):
                    pltpu.make_async_copy(
                        h_buf.at[sl], acc_sh.at[dli.at[sl]],
                        sem_s.at[sl]).wait()
            return 0

        lax.fori_loop(0, 2, chunk_body, 0)
        plsc.subcore_barrier()
        rowstart = rowbase + s * RPT

        @pl.when(rowstart + RPT <= NPAD)
        def _flush():
            pltpu.sync_copy(acc_sh.at[pl.ds(s * RPT, RPT)],
                            agg_hbm.at[pl.ds(rowstart, RPT)])

        plsc.subcore_barrier()
        return 0

    lax.fori_loop(0, NRANGE // NC, pass_body, 0)


@functools.cache
def _agg():
    return pl.kernel(
        _agg_body,
        out_type=jax.ShapeDtypeStruct((NPAD, D), _f32),
        mesh=_mesh(),
        compiler_params=pltpu.CompilerParams(
            needs_layout_passes=False, use_tc_tiling_on_sc=False),
        scratch_types=[
            pltpu.VMEM((CAP,), _i32),
            pltpu.VMEM((CAP,), _i32),
            pltpu.VMEM((CAP,), _i32),
            pltpu.VMEM((16,), _i32),
            pltpu.VMEM((NSLOT, G, D), _f32),
            pltpu.VMEM((NSLOT, G, 16), _f32),
            pltpu.VMEM((64,), _f32),
            pltpu.VMEM((NSLOT, G), _i32),
            pltpu.VMEM((2, D), _f32),
            pltpu.VMEM_SHARED((RNG, D), _f32),
            pltpu.SemaphoreType.DMA((NSLOT,)),
            pltpu.SemaphoreType.DMA((NSLOT,)),
            pltpu.SemaphoreType.DMA((NSLOT,)),
        ],
    )


def _gat_layer(x_pad, bias_in, W, a_src, a_dst, src, dst, act):
    h, s_tab, d_tab = _tc_linear_attn(x_pad, bias_in, W, a_src, a_dst, act)
    ex, denp, eidb, srcb, dstgb, cnts = _edge_a()(s_tab, d_tab, src, dst)
    w = _wmul()(ex, denp, dst)
    agg = _agg()(h, w, eidb, srcb, dstgb, cnts)
    return agg


def kernel(z, edge_index, W1, a_src1, a_dst1, b1, W2, a_src2, a_dst2, b2,
           W_lin, b_lin):
    src = edge_index[0]
    dst = edge_index[1]
    z_pad = jnp.zeros((NPAD, EMB), _f32).at[:N].set(z)
    agg1 = _gat_layer(z_pad, None, W1, a_src1, a_dst1, src, dst, act=False)
    agg2 = _gat_layer(agg1, b1, W2, a_src2, a_dst2, src, dst, act=True)
    out = _tc_final(agg2, b2, W_lin)
    return out[:N, :1] + b_lin


# speedup vs baseline: 1.0450x; 1.0450x over previous
import functools

import numpy as np
import jax
import jax.numpy as jnp
from jax import lax
from jax.experimental import pallas as pl
from jax.experimental.pallas import tpu as pltpu
from jax.experimental.pallas import tpu_sc as plsc

N = 10000
E = 320000
EMB = 128
HID = 128
HEADS = 8
D = HEADS * HID

NPAD = 10240
BM = 1024

NC = 2
NS = 16
NW = NC * NS
EC = E // NW
BLK = 80
NBLK = EC // BLK

NRANGE = 14
RNG = 768
RPT = RNG // NS
CAP = 1088
G = 16
NSLOT = 4

_i32 = jnp.int32
_f32 = jnp.float32

_SEL = np.zeros((D, 16), dtype=np.float32)
for _h in range(HEADS):
    _SEL[_h * HID:(_h + 1) * HID, _h] = 1.0

@functools.cache
def _mesh():
    return plsc.VectorSubcoreMesh(core_axis_name="c", subcore_axis_name="s",
                                  num_cores=NC, num_subcores=NS)


def _linear_attn_body(act, x_ref, b_ref, w_ref, af_src_ref, af_dst_ref,
                      sel_ref, h_ref, s_ref, d_ref):
    x = x_ref[...]
    if act:
        x = jnp.tanh(x + b_ref[...])
    h = jnp.dot(x, w_ref[...], preferred_element_type=jnp.float32)
    h_ref[...] = h
    sel = sel_ref[...]
    s_ref[...] = jnp.dot(h * af_src_ref[...], sel,
                         preferred_element_type=jnp.float32)
    d_ref[...] = jnp.dot(h * af_dst_ref[...], sel,
                         preferred_element_type=jnp.float32)


def _tc_linear_attn(x_pad, bias, W, a_src, a_dst, act):
    k = x_pad.shape[1]
    af_src = a_src.reshape(1, D)
    af_dst = a_dst.reshape(1, D)
    sel = jnp.asarray(_SEL)
    b2d = bias.reshape(1, k) if act else jnp.zeros((1, k), _f32)
    grid = NPAD // BM
    h, s, d = pl.pallas_call(
        functools.partial(_linear_attn_body, act),
        grid=(grid,),
        in_specs=[
            pl.BlockSpec((BM, k), lambda i: (i, 0)),
            pl.BlockSpec((1, k), lambda i: (0, 0)),
            pl.BlockSpec((k, D), lambda i: (0, 0)),
            pl.BlockSpec((1, D), lambda i: (0, 0)),
            pl.BlockSpec((1, D), lambda i: (0, 0)),
            pl.BlockSpec((D, 16), lambda i: (0, 0)),
        ],
        out_specs=[
            pl.BlockSpec((BM, D), lambda i: (i, 0)),
            pl.BlockSpec((BM, 16), lambda i: (i, 0)),
            pl.BlockSpec((BM, 16), lambda i: (i, 0)),
        ],
        out_shape=[
            jax.ShapeDtypeStruct((NPAD, D), _f32),
            jax.ShapeDtypeStruct((NPAD, 16), _f32),
            jax.ShapeDtypeStruct((NPAD, 16), _f32),
        ],
    )(x_pad, b2d, W, af_src, af_dst, sel)
    return h, s, d


def _recip_body(a_ref, b_ref, o_ref):
    o_ref[...] = 1.0 / (a_ref[...] + b_ref[...] + 1e-16)


def _tc_recip(denp):
    a = denp[:NPAD].reshape(1280, 128)
    b = denp[NPAD:].reshape(1280, 128)
    out = pl.pallas_call(
        _recip_body,
        out_shape=jax.ShapeDtypeStruct((1280, 128), _f32),
    )(a, b)
    return out.reshape(NPAD, 16)


def _final_body(x_ref, b_ref, wl_ref, o_ref):
    x = jnp.tanh(x_ref[...] + b_ref[...])
    o_ref[...] = jnp.dot(x, wl_ref[...], preferred_element_type=jnp.float32)


def _tc_final(pre, bias, W_lin):
    wl = jnp.zeros((D, 128), _f32).at[:, :1].set(W_lin)
    b2d = bias.reshape(1, D)
    out = pl.pallas_call(
        _final_body,
        grid=(NPAD // BM,),
        in_specs=[
            pl.BlockSpec((BM, D), lambda i: (i, 0)),
            pl.BlockSpec((1, D), lambda i: (0, 0)),
            pl.BlockSpec((D, 128), lambda i: (0, 0)),
        ],
        out_specs=pl.BlockSpec((BM, 128), lambda i: (i, 0)),
        out_shape=jax.ShapeDtypeStruct((NPAD, 128), _f32),
    )(pre, b2d, wl)
    return out


def _edge_a_body(s_tab, d_tab, src_hbm, dst_hbm,
                 ex_hbm, denp_hbm, eid_hbm, srcb_hbm, dstgb_hbm, cnt_hbm,
                 src_v, dst_v, s_rows, d_rows, bk_eid, bk_src, bk_dstg,
                 zeros_v, idx_scr, dsti, den_sh, sem_g1, sem_g2, sem_f,
                 sem_e):
    c = lax.axis_index("c")
    s = lax.axis_index("s")
    wid = s * NC + c
    ebase = wid * EC

    pltpu.sync_copy(src_hbm.at[pl.ds(ebase, EC)], src_v)
    pltpu.sync_copy(dst_hbm.at[pl.ds(ebase, EC)], dst_v)

    zvec = jnp.zeros((16,), _f32)
    for i in range(128):
        zeros_v[i, :] = zvec
    for r in range(5):
        pltpu.sync_copy(zeros_v, den_sh.at[pl.ds(s * 640 + r * 128, 128)])
    plsc.subcore_barrier()

    lane = lax.iota(_i32, 16)

    def do_block(i, sl, offs):
        eb = i * BLK
        pltpu.make_async_copy(s_tab.at[src_v.at[pl.ds(eb, BLK)]],
                              s_rows.at[sl], sem_g1.at[sl]).wait()
        pltpu.make_async_copy(d_tab.at[dst_v.at[pl.ds(eb, BLK)]],
                              d_rows.at[sl], sem_g2.at[sl]).wait()

        def sub(st, offs):
            sb = st * 16
            for r in range(16):
                idx = sb + r
                ev = s_rows[sl, idx, :] + d_rows[sl, idx, :]
                ev = jnp.where(ev >= 0.0, ev, 0.2 * ev)
                s_rows[sl, idx, :] = jnp.exp(ev)
            dstv = dst_v[pl.ds(eb + sb, 16)]
            dsti[sl, pl.ds(sb, 16)] = dstv
            srcv = src_v[pl.ds(eb + sb, 16)]
            eidv = jnp.full((16,), ebase + eb + sb, _i32) + lane
            new_offs = []
            for b in range(NRANGE):
                lo = b * RNG
                m = (dstv >= lo) & (dstv < lo + RNG)
                cnt = jnp.max(plsc.all_reduce_population_count(m))
                rel = offs[b]
                addr = b * CAP + rel
                plsc.store_compressed(bk_eid.at[pl.ds(addr, 16)], eidv,
                                      mask=m)
                plsc.store_compressed(bk_src.at[pl.ds(addr, 16)], srcv,
                                      mask=m)
                plsc.store_compressed(bk_dstg.at[pl.ds(addr, 16)], dstv,
                                      mask=m)
                new_offs.append(jnp.minimum(rel + cnt, CAP - 64))
            return tuple(new_offs)

        offs = lax.fori_loop(0, BLK // 16, sub, offs)
        pltpu.async_copy(s_rows.at[sl], den_sh.at[dsti.at[sl]],
                         sem_f.at[sl], add=True)
        pltpu.async_copy(s_rows.at[sl], ex_hbm.at[pl.ds(ebase + eb, BLK)],
                         sem_e.at[sl])
        return offs

    def issue_blk(j, sl):
        eb = j * BLK
        pltpu.async_copy(s_tab.at[src_v.at[pl.ds(eb, BLK)]],
                         s_rows.at[sl], sem_g1.at[sl])
        pltpu.async_copy(d_tab.at[dst_v.at[pl.ds(eb, BLK)]],
                         d_rows.at[sl], sem_g2.at[sl])

    def wait_flush(sl):
        pltpu.make_async_copy(s_rows.at[sl], den_sh.at[dsti.at[sl]],
                              sem_f.at[sl]).wait()
        pltpu.make_async_copy(s_rows.at[sl], ex_hbm.at[pl.ds(0, BLK)],
                              sem_e.at[sl]).wait()

    issue_blk(0, 0)
    issue_blk(1, 1)

    def group(i3, offs):
        for sl in range(3):
            i = i3 * 3 + sl
            offs = do_block(i, sl, offs)
            so2 = (sl + 2) % 3

            @pl.when(i >= 1)
            def _wf(so2=so2):
                wait_flush(so2)

            issue_blk(i + 2, so2)
        return offs

    offs = lax.fori_loop(0, (NBLK - 2) // 3, group,
                         tuple(jnp.zeros((), _i32) for _ in range(NRANGE)))
    offs = do_block(jnp.int32(NBLK - 2), 0, offs)
    offs = do_block(jnp.int32(NBLK - 1), 1, offs)
    for sl in (2, 0, 1):
        wait_flush(sl)

    zi = jnp.zeros((16,), _i32)
    ze = jnp.full((16,), E, _i32)
    for b in range(NRANGE):
        padd = jnp.full((16,), b * RNG, _i32)
        for t in range(3):
            o = b * CAP + offs[b] + t * 16
            bk_eid[pl.ds(o, 16)] = ze
            bk_src[pl.ds(o, 16)] = zi
            bk_dstg[pl.ds(o, 16)] = padd
    cv = jnp.zeros((16,), _i32)
    for b in range(NRANGE):
        cv = jnp.where(lane == b, jnp.full((16,), 1, _i32) * offs[b], cv)
    idx_scr[...] = cv
    pltpu.sync_copy(idx_scr, cnt_hbm.at[pl.ds(wid * 16, 16)])
    for b in range(NRANGE):
        seg = (wid * NRANGE + b) * CAP
        pltpu.sync_copy(bk_eid.at[pl.ds(b * CAP, CAP)],
                        eid_hbm.at[pl.ds(seg, CAP)])
        pltpu.sync_copy(bk_src.at[pl.ds(b * CAP, CAP)],
                        srcb_hbm.at[pl.ds(seg, CAP)])
        pltpu.sync_copy(bk_dstg.at[pl.ds(b * CAP, CAP)],
                        dstgb_hbm.at[pl.ds(seg, CAP)])

    plsc.subcore_barrier()
    pltpu.sync_copy(den_sh.at[pl.ds(s * 640, 640)],
                    denp_hbm.at[pl.ds(c * NPAD + s * 640, 640)])


@functools.cache
def _edge_a():
    return pl.kernel(
        _edge_a_body,
        out_type=[
            jax.ShapeDtypeStruct((E, 16), _f32),
            jax.ShapeDtypeStruct((NC * NPAD, 16), _f32),
            jax.ShapeDtypeStruct((NW * NRANGE * CAP,), _i32),
            jax.ShapeDtypeStruct((NW * NRANGE * CAP,), _i32),
            jax.ShapeDtypeStruct((NW * NRANGE * CAP,), _i32),
            jax.ShapeDtypeStruct((NW * 16,), _i32),
        ],
        mesh=_mesh(),
        compiler_params=pltpu.CompilerParams(
            needs_layout_passes=False, use_tc_tiling_on_sc=False),
        scratch_types=[
            pltpu.VMEM((EC,), _i32),
            pltpu.VMEM((EC,), _i32),
            pltpu.VMEM((3, BLK, 16), _f32),
            pltpu.VMEM((3, BLK, 16), _f32),
            pltpu.VMEM((NRANGE * CAP,), _i32),
            pltpu.VMEM((NRANGE * CAP,), _i32),
            pltpu.VMEM((NRANGE * CAP,), _i32),
            pltpu.VMEM((128, 16), _f32),
            pltpu.VMEM((16,), _i32),
            pltpu.VMEM((3, BLK), _i32),
            pltpu.VMEM_SHARED((NPAD, 16), _f32),
            pltpu.SemaphoreType.DMA((3,)),
            pltpu.SemaphoreType.DMA((3,)),
            pltpu.SemaphoreType.DMA((3,)),
            pltpu.SemaphoreType.DMA((3,)),
        ],
    )


def _wmul_body(ex_hbm, denp_hbm, dst_hbm, w_hbm, exb, dnb, dn2, dst_v,
               dst2_v, sem1, sem2, sem3, sem_w):
    c = lax.axis_index("c")
    s = lax.axis_index("s")
    wid = s * NC + c
    ebase = wid * EC
    pltpu.sync_copy(dst_hbm.at[pl.ds(ebase, EC)], dst_v)
    npadv = jnp.full((16,), NPAD, _i32)

    def shift(i, _):
        dst2_v[pl.ds(i * 16, 16)] = dst_v[pl.ds(i * 16, 16)] + npadv
        return 0

    lax.fori_loop(0, EC // 16, shift, 0)

    def issue_b(j, sl):
        eb = j * BLK
        pltpu.async_copy(ex_hbm.at[pl.ds(ebase + eb, BLK)],
                         exb.at[sl], sem1.at[sl])
        pltpu.async_copy(denp_hbm.at[dst_v.at[pl.ds(eb, BLK)]],
                         dnb.at[sl], sem2.at[sl])
        pltpu.async_copy(denp_hbm.at[dst2_v.at[pl.ds(eb, BLK)]],
                         dn2.at[sl], sem3.at[sl])

    issue_b(0, 0)
    issue_b(1, 1)

    def bgroup(i3, _):
        for sl in range(3):
            i = i3 * 3 + sl

            @pl.when(i < NBLK)
            def _do(i=i, sl=sl):
                eb = i * BLK
                pltpu.make_async_copy(
                    ex_hbm.at[pl.ds(ebase + eb, BLK)],
                    exb.at[sl], sem1.at[sl]).wait()
                pltpu.make_async_copy(
                    denp_hbm.at[dst_v.at[pl.ds(eb, BLK)]],
                    dnb.at[sl], sem2.at[sl]).wait()
                pltpu.make_async_copy(
                    denp_hbm.at[dst2_v.at[pl.ds(eb, BLK)]],
                    dn2.at[sl], sem3.at[sl]).wait()
                for r in range(BLK):
                    exb[sl, r, :] = (exb[sl, r, :]
                                     / (dnb[sl, r, :] + dn2[sl, r, :]
                                        + 1e-16))
                pltpu.async_copy(exb.at[sl],
                                 w_hbm.at[pl.ds(ebase + eb, BLK)],
                                 sem_w.at[sl])
                jn = i + 2
                so2 = (sl + 2) % 3

                @pl.when(jn < NBLK)
                def _prep():
                    @pl.when(i >= 1)
                    def _wp():
                        pltpu.make_async_copy(
                            exb.at[so2],
                            w_hbm.at[pl.ds(0, BLK)],
                            sem_w.at[so2]).wait()
                    issue_b(jn, so2)
        return 0

    lax.fori_loop(0, (NBLK + 2) // 3, bgroup, 0)
    for sl in range(3):
        pltpu.make_async_copy(exb.at[sl], w_hbm.at[pl.ds(0, BLK)],
                              sem_w.at[sl]).wait()

    @pl.when(wid == 0)
    def _zero_tail():
        for r in range(16):
            dnb[0, r, :] = jnp.zeros((16,), _f32)
        pltpu.sync_copy(dnb.at[0, pl.ds(0, 16)], w_hbm.at[pl.ds(E, 16)])


@functools.cache
def _wmul():
    return pl.kernel(
        _wmul_body,
        out_type=jax.ShapeDtypeStruct((E + 16, 16), _f32),
        mesh=_mesh(),
        compiler_params=pltpu.CompilerParams(
            needs_layout_passes=False, use_tc_tiling_on_sc=False),
        scratch_types=[
            pltpu.VMEM((3, BLK, 16), _f32),
            pltpu.VMEM((3, BLK, 16), _f32),
            pltpu.VMEM((3, BLK, 16), _f32),
            pltpu.VMEM((EC,), _i32),
            pltpu.VMEM((EC,), _i32),
            pltpu.SemaphoreType.DMA((3,)),
            pltpu.SemaphoreType.DMA((3,)),
            pltpu.SemaphoreType.DMA((3,)),
            pltpu.SemaphoreType.DMA((3,)),
        ],
    )


def _agg_body(h_hbm, w_hbm, eidb_hbm, srcb_hbm, dstgb_hbm,
              cnt_hbm, agg_hbm,
              eid_v, src_v, dstg_v, cnt_v, h_buf, exw, w_scr, dli,
              zer, acc_sh, sem_gh, sem_ge, sem_s):
    c = lax.axis_index("c")
    s = lax.axis_index("s")
    lane = lax.iota(_i32, 16)

    zvec = jnp.zeros((16,), _f32)
    for i in range(2):
        for j in range(64):
            zer[i, pl.ds(j * 16, 16)] = zvec

    def pass_body(p, _):
        b = p * NC + c
        rowbase = b * RNG

        def zero_body(r, _):
            pltpu.sync_copy(zer, acc_sh.at[pl.ds(s * RPT + r * 2, 2)])
            return 0

        lax.fori_loop(0, RPT // 2, zero_body, 0)
        plsc.subcore_barrier()

        def chunk_body(slot, _):
            chunk = s * 2 + slot
            seg = (chunk * NRANGE + b) * CAP
            pltpu.sync_copy(eidb_hbm.at[pl.ds(seg, CAP)], eid_v)
            pltpu.sync_copy(srcb_hbm.at[pl.ds(seg, CAP)], src_v)
            pltpu.sync_copy(dstgb_hbm.at[pl.ds(seg, CAP)], dstg_v)
            pltpu.sync_copy(cnt_hbm.at[pl.ds(chunk * 16, 16)], cnt_v)
            bspl = jnp.full((16,), 1, _i32) * b
            count = jnp.max(jnp.where(lane == bspl, cnt_v[...], 0))
            nb = (count + (G - 1)) >> 4
            rb = jnp.full((16,), 1, _i32) * rowbase

            def issue(j, sl):
                base = j * G
                pltpu.async_copy(h_hbm.at[src_v.at[pl.ds(base, G)]],
                                 h_buf.at[sl], sem_gh.at[sl])
                pltpu.async_copy(w_hbm.at[eid_v.at[pl.ds(base, G)]],
                                 exw.at[sl], sem_ge.at[sl])

            @pl.when(nb > 0)
            def _prime0():
                issue(0, 0)

            @pl.when(nb > 1)
            def _prime1():
                issue(1, 1)

            @pl.when(nb > 2)
            def _prime2():
                issue(2, 2)

            def group(i3, _):
                for sl in range(NSLOT):
                    i = i3 * NSLOT + sl

                    @pl.when(i < nb)
                    def _do(i=i, sl=sl):
                        base = i * G
                        pltpu.make_async_copy(
                            h_hbm.at[src_v.at[pl.ds(base, G)]],
                            h_buf.at[sl], sem_gh.at[sl]).wait()
                        pltpu.make_async_copy(
                            w_hbm.at[eid_v.at[pl.ds(base, G)]],
                            exw.at[sl], sem_ge.at[sl]).wait()
                        dli[sl, pl.ds(0, 16)] = (
                            dstg_v[pl.ds(base, 16)] - rb)

                        def edge(g2, _):
                            for e in range(2):
                                g = g2 * 2 + e
                                wv = exw[sl, g, :]
                                w_scr[pl.ds(e * 32, 16)] = wv
                                w_scr[pl.ds(e * 32 + 16, 16)] = wv
                            for e in range(2):
                                g = g2 * 2 + e
                                for hh in range(HEADS):
                                    spl = plsc.load_gather(
                                        w_scr,
                                        [jnp.full((16,),
                                                  e * 32 + 16 + hh, _i32)])
                                    for jj in range(8):
                                        off = hh * HID + jj * 16
                                        h_buf[sl, g, pl.ds(off, 16)] = (
                                            h_buf[sl, g, pl.ds(off, 16)]
                                            * spl)
                            return 0

                        lax.fori_loop(0, G // 2, edge, 0)
                        pltpu.async_copy(h_buf.at[sl],
                                         acc_sh.at[dli.at[sl]],
                                         sem_s.at[sl], add=True)
                        jn = i + 3
                        so2 = (sl + 3) % NSLOT

                        @pl.when(jn < nb)
                        def _prep():
                            @pl.when(i >= 1)
                            def _wait_prev():
                                pltpu.make_async_copy(
                                    h_buf.at[so2],
                                    acc_sh.at[dli.at[so2]],
                                    sem_s.at[so2]).wait()
                            issue(jn, so2)
                return 0

            nb3 = (nb + (NSLOT - 1)) // NSLOT
            lax.fori_loop(0, nb3, group, 0)
            for sl in range(NSLOT):
                @pl.when(sl < nb)
                def _drain(sl=sl):
                    pltpu.make_async_copy(
                        h_buf.at[sl], acc_sh.at[dli.at[sl]],
                        sem_s.at[sl]).wait()
            return 0

        lax.fori_loop(0, 2, chunk_body, 0)
        plsc.subcore_barrier()
        rowstart = rowbase + s * RPT

        @pl.when(rowstart + RPT <= NPAD)
        def _flush():
            pltpu.sync_copy(acc_sh.at[pl.ds(s * RPT, RPT)],
                            agg_hbm.at[pl.ds(rowstart, RPT)])

        plsc.subcore_barrier()
        return 0

    lax.fori_loop(0, NRANGE // NC, pass_body, 0)


@functools.cache
def _agg():
    return pl.kernel(
        _agg_body,
        out_type=jax.ShapeDtypeStruct((NPAD, D), _f32),
        mesh=_mesh(),
        compiler_params=pltpu.CompilerParams(
            needs_layout_passes=False, use_tc_tiling_on_sc=False),
        scratch_types=[
            pltpu.VMEM((CAP,), _i32),
            pltpu.VMEM((CAP,), _i32),
            pltpu.VMEM((CAP,), _i32),
            pltpu.VMEM((16,), _i32),
            pltpu.VMEM((NSLOT, G, D), _f32),
            pltpu.VMEM((NSLOT, G, 16), _f32),
            pltpu.VMEM((64,), _f32),
            pltpu.VMEM((NSLOT, G), _i32),
            pltpu.VMEM((2, D), _f32),
            pltpu.VMEM_SHARED((RNG, D), _f32),
            pltpu.SemaphoreType.DMA((NSLOT,)),
            pltpu.SemaphoreType.DMA((NSLOT,)),
            pltpu.SemaphoreType.DMA((NSLOT,)),
        ],
    )


def _gat_layer(x_pad, bias_in, W, a_src, a_dst, src, dst, act):
    h, s_tab, d_tab = _tc_linear_attn(x_pad, bias_in, W, a_src, a_dst, act)
    ex, denp, eidb, srcb, dstgb, cnts = _edge_a()(s_tab, d_tab, src, dst)
    w = _wmul()(ex, denp, dst)
    agg = _agg()(h, w, eidb, srcb, dstgb, cnts)
    return agg


def kernel(z, edge_index, W1, a_src1, a_dst1, b1, W2, a_src2, a_dst2, b2,
           W_lin, b_lin):
    src = edge_index[0]
    dst = edge_index[1]
    z_pad = jnp.zeros((NPAD, EMB), _f32).at[:N].set(z)
    agg1 = _gat_layer(z_pad, None, W1, a_src1, a_dst1, src, dst, act=False)
    agg2 = _gat_layer(agg1, b1, W2, a_src2, a_dst2, src, dst, act=True)
    out = _tc_final(agg2, b2, W_lin)
    return out[:N, :1] + b_lin
